# baseline (speedup 1.0000x reference)
import functools

import jax
import jax.numpy as jnp
from jax import lax
from jax.experimental import pallas as pl
from jax.experimental.pallas import tpu as pltpu
from jax.experimental.pallas import tpu_sc as plsc

BATCH = 16384
HIST = 200
SIZE = 16
NC = 2
NS = 16
NW = NC * NS
HG = HIST // 8
BG = BATCH // 128
NCH_TOT = HG * BG
CPW = NCH_TOT // NW
CL = 8 * 128
NBUF = 2
assert CPW % NBUF == 0


def _make_gather():
    mesh = plsc.VectorSubcoreMesh(core_axis_name="c", subcore_axis_name="s")

    @functools.partial(
        pl.kernel,
        mesh=mesh,
        out_type=jax.ShapeDtypeStruct((HIST, 2, BG, 8, 128), jnp.float32),
        scratch_types=[
            pltpu.VMEM((NBUF, CL), jnp.int32),
            pltpu.VMEM((NBUF, CL, SIZE), jnp.float32),
            pltpu.VMEM((NBUF, 8, 2, 8, 128), jnp.float32),
            [pltpu.SemaphoreType.DMA] * NBUF,
            [pltpu.SemaphoreType.DMA] * NBUF,
            [pltpu.SemaphoreType.DMA] * NBUF,
        ],
        compiler_params=pltpu.CompilerParams(
            use_tc_tiling_on_sc=False, needs_layout_passes=False,
            disable_bounds_checks=True),
    )
    def gather_kernel(idx_hbm, table_hbm, out_hbm, idx_v, rows_v, t_v,
                      sem_i, sem_g, sem_o):
        wid = lax.axis_index("s") * NC + lax.axis_index("c")
        c0 = wid * CPW

        iota = lax.iota(jnp.int32, 16)
        dcol = [jnp.full((16,), d, jnp.int32) for d in range(SIZE)]

        def idx_copies(c, b):
            hg = c // BG
            bg = lax.rem(c, BG)
            return [pltpu.make_async_copy(
                        idx_hbm.at[hg, bg, hl],
                        idx_v.at[b, pl.ds(hl * 128, 128)], sem_i[b])
                    for hl in range(8)]

        def gather(b):
            return pltpu.make_async_copy(
                table_hbm.at[idx_v.at[b]], rows_v.at[b], sem_g[b])

        def out_stores(c, b):
            hg = c // BG
            bg = lax.rem(c, BG)
            return [pltpu.make_async_copy(
                        t_v.at[b, hl],
                        out_hbm.at[hg * 8 + hl, :, bg], sem_o[b])
                    for hl in range(8)]

        def start(ds):
            for d in ds:
                d.start()

        def wait(ds):
            for d in ds:
                d.wait()

        def transpose(b):
            rows = rows_v.at[b]

            @pl.loop(0, 8)
            def _hl(hl):
                base = hl * 128
                for blb in range(8):
                    ridx = iota + (base + blb * 16)
                    for d in range(SIZE):
                        v = plsc.load_gather(rows, [ridx, dcol[d]])
                        t_v[b, hl, d // 8, d % 8,
                            pl.ds(blb * 16, 16)] = v

        def retire(c, b, prefetch):
            gather(b).wait()
            transpose(b)
            if prefetch:
                @pl.when(c + NBUF < CPW)
                def _():
                    start(idx_copies(c0 + c + NBUF, b))
            start(out_stores(c0 + c, b))

        for b in range(NBUF):
            start(idx_copies(c0 + b, b))

        @pl.loop(0, CPW, step=NBUF)
        def _chunks(g0):
            for b in range(NBUF):
                g = g0 + b
                pb = (b - 1) % NBUF
                wait(idx_copies(c0 + g, b))

                @pl.when(g >= NBUF)
                def _():
                    wait(out_stores(c0 + g - NBUF, b))

                gather(b).start()

                @pl.when(g >= 1)
                def _():
                    retire(g - 1, pb, prefetch=True)

        retire(CPW - 1, (CPW - 1) % NBUF, prefetch=False)
        for g in range(CPW - NBUF, CPW):
            wait(out_stores(c0 + g, g % NBUF))

    return gather_kernel


_gather = _make_gather()


@jax.jit
def kernel(x, W):
    x4 = (x.astype(jnp.int32).T
          .reshape(HG, 8, BG, 128).transpose(0, 2, 1, 3))
    out5 = _gather(x4, W)
    return (out5.transpose(0, 1, 3, 2, 4)
            .reshape(HIST, SIZE, BATCH).transpose(2, 0, 1))


# speedup vs baseline: 1.9909x; 1.9909x over previous
import functools

import jax
import jax.numpy as jnp
from jax import lax
from jax.experimental import pallas as pl
from jax.experimental.pallas import tpu as pltpu
from jax.experimental.pallas import tpu_sc as plsc

BATCH = 16384
HIST = 200
SIZE = 16
NC = 2
NS = 16
NW = NC * NS
HG = HIST // 8
BG = BATCH // 128
NCH_TOT = HG * BG
CPW = NCH_TOT // NW
CL = 8 * 128
NBUF = 2
assert CPW % NBUF == 0


def _make_gather():
    mesh = plsc.VectorSubcoreMesh(core_axis_name="c", subcore_axis_name="s")

    @functools.partial(
        pl.kernel,
        mesh=mesh,
        out_type=jax.ShapeDtypeStruct((HIST, 2, BG, 8, 128), jnp.float32),
        scratch_types=[
            pltpu.VMEM((NBUF, CL), jnp.int32),
            pltpu.VMEM((NBUF, CL, SIZE), jnp.float32),
            pltpu.VMEM((NBUF, 8, 2, 8, 128), jnp.float32),
            [pltpu.SemaphoreType.DMA] * NBUF,
            [pltpu.SemaphoreType.DMA] * NBUF,
            [pltpu.SemaphoreType.DMA] * NBUF,
        ],
        compiler_params=pltpu.CompilerParams(
            use_tc_tiling_on_sc=False, needs_layout_passes=False,
            disable_bounds_checks=True),
    )
    def gather_kernel(idx_hbm, table_hbm, out_hbm, idx_v, rows_v, t_v,
                      sem_i, sem_g, sem_o):
        wid = lax.axis_index("s") * NC + lax.axis_index("c")
        c0 = wid * CPW

        iota = lax.iota(jnp.int32, 16)
        dcol = [jnp.full((16,), d, jnp.int32) for d in range(SIZE)]

        def idx_copies(c, b):
            hg = c // BG
            bg = lax.rem(c, BG)
            return [pltpu.make_async_copy(
                        idx_hbm.at[hg, bg, hl],
                        idx_v.at[b, pl.ds(hl * 128, 128)], sem_i[b])
                    for hl in range(8)]

        def gather(b):
            return pltpu.make_async_copy(
                table_hbm.at[idx_v.at[b]], rows_v.at[b], sem_g[b])

        def out_stores(c, b):
            hg = c // BG
            bg = lax.rem(c, BG)
            return [pltpu.make_async_copy(
                        t_v.at[b, hl],
                        out_hbm.at[hg * 8 + hl, :, bg], sem_o[b])
                    for hl in range(8)]

        def start(ds):
            for d in ds:
                d.start()

        def wait(ds):
            for d in ds:
                d.wait()

        def transpose(b):
            rows = rows_v.at[b]

            @functools.partial(plsc.parallel_loop, 0, 8, unroll=2)
            def _hl(hl):
                base = hl * 128
                for blb in range(8):
                    ridx = iota + (base + blb * 16)
                    for d in range(SIZE):
                        v = plsc.load_gather(rows, [ridx, dcol[d]])
                        t_v[b, hl, d // 8, d % 8,
                            pl.ds(blb * 16, 16)] = v
            plsc.subcore_barrier()

        def retire(c, b, prefetch):
            gather(b).wait()
            transpose(b)
            if prefetch:
                @pl.when(c + NBUF < CPW)
                def _():
                    start(idx_copies(c0 + c + NBUF, b))
            start(out_stores(c0 + c, b))

        for b in range(NBUF):
            start(idx_copies(c0 + b, b))

        @pl.loop(0, CPW, step=NBUF)
        def _chunks(g0):
            for b in range(NBUF):
                g = g0 + b
                pb = (b - 1) % NBUF
                wait(idx_copies(c0 + g, b))

                @pl.when(g >= NBUF)
                def _():
                    wait(out_stores(c0 + g - NBUF, b))

                gather(b).start()

                @pl.when(g >= 1)
                def _():
                    retire(g - 1, pb, prefetch=True)

        retire(CPW - 1, (CPW - 1) % NBUF, prefetch=False)
        for g in range(CPW - NBUF, CPW):
            wait(out_stores(c0 + g, g % NBUF))

    return gather_kernel


_gather = _make_gather()


@jax.jit
def kernel(x, W):
    x4 = (x.astype(jnp.int32).T
          .reshape(HG, 8, BG, 128).transpose(0, 2, 1, 3))
    out5 = _gather(x4, W)
    return (out5.transpose(0, 1, 3, 2, 4)
            .reshape(HIST, SIZE, BATCH).transpose(2, 0, 1))
